# baseline (speedup 1.0000x reference)
import functools

import jax
import jax.numpy as jnp
import numpy as np
from jax import lax
from jax.experimental import pallas as pl
from jax.experimental.pallas import tpu as pltpu
from jax.experimental.pallas import tpu_sc as plsc

N = 1000000
D = 16
NC = 2
NS = 16
NW = NC * NS
L = 16

NP = 1 << 20
SEG = NP // NW
VSEG = SEG // L

BITS = (11, 11, 10)
SHIFTS = (0, 11, 22)
SIGN = np.int32(-2147483648)

_MESH = plsc.VectorSubcoreMesh(core_axis_name="c", subcore_axis_name="s",
                               num_cores=NC, num_subcores=NS)
_SC_PARAMS = pltpu.CompilerParams(use_tc_tiling_on_sc=False,
                                  needs_layout_passes=False)


def _wid():
    return lax.axis_index("s") * NC + lax.axis_index("c")


def _to_key(b):
    return jnp.where(b < 0, ~b, b | SIGN)


def _from_key(k):
    return plsc.bitcast(jnp.where(k < 0, k ^ SIGN, ~k), jnp.float32)


def _digit(k, shift, mask):
    return lax.shift_right_logical(k, np.int32(shift)) & np.int32(mask)


def _make_hist(shift, nbits, first_pass):
    r = 1 << nbits
    mask = r - 1

    @functools.partial(
        pl.kernel,
        out_type=jax.ShapeDtypeStruct((NW * r,), jnp.int32),
        mesh=_MESH,
        scratch_types=(
            pltpu.VMEM((SEG,), jnp.int32),
            pltpu.VMEM((r,), jnp.int32),
            pltpu.SemaphoreType.DMA,
        ),
        compiler_params=_SC_PARAMS,
    )
    def hist_kernel(key_hbm, hist_hbm, key_v, hist_v, sem):
        w = _wid()
        zeros = lax.full((L,), np.int32(0), jnp.int32)
        for i in range(r // L):
            hist_v[pl.ds(i * L, L)] = zeros
        pltpu.sync_copy(key_hbm.at[pl.ds(w * SEG, SEG)], key_v)

        def body(i, _):
            k = key_v[pl.ds(i * L, L)]
            if first_pass:
                k = _to_key(k)
            d = _digit(k, shift, mask)
            cnt, last = plsc.scan_count(d)
            plsc.addupdate_scatter(hist_v, [d], cnt, mask=last)
            return 0

        lax.fori_loop(0, VSEG, body, 0)
        pltpu.sync_copy(hist_v, hist_hbm.at[pl.ds(w * r, r)])

    return hist_kernel


def _make_scan(nbits):
    r = 1 << nbits
    dpw = r // NW

    @functools.partial(
        pl.kernel,
        out_type=(
            jax.ShapeDtypeStruct((r * NW,), jnp.int32),
            jax.ShapeDtypeStruct((r,), jnp.int32),
        ),
        mesh=_MESH,
        scratch_types=(
            pltpu.VMEM((NW * dpw,), jnp.int32),
            pltpu.VMEM((dpw * NW,), jnp.int32),
            pltpu.VMEM((dpw,), jnp.int32),
            pltpu.SemaphoreType.DMA,
        ),
        compiler_params=_SC_PARAMS,
    )
    def scan_kernel(hist_hbm, prefix_hbm, tot_hbm, rows_v, pref_v, tot_v, sem):
        w = _wid()
        for wp in range(NW):
            pltpu.sync_copy(
                hist_hbm.at[pl.ds(wp * r + w * dpw, dpw)],
                rows_v.at[pl.ds(wp * dpw, dpw)],
            )
        dl16 = lax.iota(jnp.int32, L)

        def dblock(i, _):
            dbase = i * L
            acc0 = lax.full((L,), np.int32(0), jnp.int32)

            def wloop(wp, acc):
                c = plsc.load_gather(rows_v, [wp * dpw + dbase + dl16])
                plsc.store_scatter(pref_v, [(dbase + dl16) * NW + wp], acc)
                return acc + c

            acc = lax.fori_loop(0, NW, wloop, acc0)
            tot_v[pl.ds(dbase, L)] = acc
            return 0

        lax.fori_loop(0, dpw // L, dblock, 0)
        pltpu.sync_copy(pref_v, prefix_hbm.at[pl.ds(w * dpw * NW, dpw * NW)])
        pltpu.sync_copy(tot_v, tot_hbm.at[pl.ds(w * dpw, dpw)])

    return scan_kernel


def _make_permute(shift, nbits, first_pass):
    r = 1 << nbits
    mask = r - 1

    scratch = (
        pltpu.VMEM((SEG,), jnp.int32),
        pltpu.VMEM((SEG,), jnp.int32),
        pltpu.VMEM((SEG,), jnp.int32),
        pltpu.VMEM((r,), jnp.int32),
        pltpu.VMEM((r,), jnp.int32),
        pltpu.VMEM((r,), jnp.int32),
        pltpu.SemaphoreType.DMA,
    )
    out_type = (
        jax.ShapeDtypeStruct((NP,), jnp.int32),
        jax.ShapeDtypeStruct((NP,), jnp.int32),
    )

    def body_common(w, key_hbm, prefix_hbm, tot_hbm, kout_hbm, iout_hbm,
                    key_v, idx_v, pos_v, tot_v, col_v, ctr_v, sem):
        pltpu.sync_copy(key_hbm.at[pl.ds(w * SEG, SEG)], key_v)
        pltpu.sync_copy(tot_hbm, tot_v)

        def mkidx(i, _):
            ctr_v[pl.ds(i * L, L)] = (i * L + lax.iota(jnp.int32, L)) * NW + w
            return 0
        lax.fori_loop(0, r // L, mkidx, 0)
        pltpu.async_copy(prefix_hbm.at[ctr_v], col_v, sem).wait()

        def scan_step(i, carry):
            t = tot_v[pl.ds(i * L, L)]
            inc = plsc.cumsum(t)
            ctr_v[pl.ds(i * L, L)] = inc - t + carry + col_v[pl.ds(i * L, L)]
            return carry + jnp.sum(t)
        lax.fori_loop(0, r // L, scan_step, np.int32(0))

        def body(i, _):
            k = key_v[pl.ds(i * L, L)]
            if first_pass:
                k = _to_key(k)
                key_v[pl.ds(i * L, L)] = k
            d = _digit(k, shift, mask)
            cnt, last = plsc.scan_count(d)
            cur = plsc.load_gather(ctr_v, [d])
            pos_v[pl.ds(i * L, L)] = cur + cnt - 1
            plsc.addupdate_scatter(ctr_v, [d], cnt, mask=last)
            return 0
        lax.fori_loop(0, VSEG, body, 0)

        pltpu.async_copy(key_v, kout_hbm.at[pos_v], sem).wait()
        pltpu.async_copy(idx_v, iout_hbm.at[pos_v], sem).wait()

    if first_pass:
        @functools.partial(
            pl.kernel, out_type=out_type, mesh=_MESH,
            scratch_types=scratch, compiler_params=_SC_PARAMS,
        )
        def permute_kernel(key_hbm, prefix_hbm, tot_hbm, kout_hbm, iout_hbm,
                           key_v, idx_v, pos_v, tot_v, col_v, ctr_v, sem):
            w = _wid()

            def gen(i, _):
                idx_v[pl.ds(i * L, L)] = (
                    w * SEG + i * L + lax.iota(jnp.int32, L))
                return 0
            lax.fori_loop(0, VSEG, gen, 0)
            body_common(w, key_hbm, prefix_hbm, tot_hbm, kout_hbm, iout_hbm,
                        key_v, idx_v, pos_v, tot_v, col_v, ctr_v, sem)
    else:
        @functools.partial(
            pl.kernel, out_type=out_type, mesh=_MESH,
            scratch_types=scratch, compiler_params=_SC_PARAMS,
        )
        def permute_kernel(key_hbm, idx_hbm, prefix_hbm, tot_hbm,
                           kout_hbm, iout_hbm,
                           key_v, idx_v, pos_v, tot_v, col_v, ctr_v, sem):
            w = _wid()
            pltpu.sync_copy(idx_hbm.at[pl.ds(w * SEG, SEG)], idx_v)
            body_common(w, key_hbm, prefix_hbm, tot_hbm, kout_hbm, iout_hbm,
                        key_v, idx_v, pos_v, tot_v, col_v, ctr_v, sem)

    return permute_kernel


CHUNK = 2048
NFULL = N // CHUNK
TAIL = N - NFULL * CHUNK
JMAX = (NFULL + NW) // NW


@functools.partial(
    pl.kernel,
    out_type=(
        jax.ShapeDtypeStruct((N, D), jnp.float32),
        jax.ShapeDtypeStruct((N,), jnp.float32),
    ),
    mesh=_MESH,
    scratch_types=(
        pltpu.VMEM((CHUNK,), jnp.int32),
        pltpu.VMEM((CHUNK, D), jnp.float32),
        pltpu.VMEM((CHUNK,), jnp.int32),
        pltpu.VMEM((CHUNK,), jnp.float32),
        pltpu.SemaphoreType.DMA,
    ),
    compiler_params=_SC_PARAMS,
)
def _gather_kernel(x_hbm, key_hbm, idx_hbm, xs_hbm, fs_hbm,
                   idx_v, rows_v, key_v, fit_v, sem):
    w = _wid()
    for j in range(JMAX):
        c = w + j * NW
        off = c * CHUNK

        def do(n):
            pltpu.sync_copy(idx_hbm.at[pl.ds(off, n)],
                            idx_v.at[pl.ds(0, n)])
            pltpu.async_copy(x_hbm.at[idx_v.at[pl.ds(0, n)]],
                             rows_v.at[pl.ds(0, n)], sem).wait()
            pltpu.sync_copy(rows_v.at[pl.ds(0, n)],
                            xs_hbm.at[pl.ds(off, n)])
            pltpu.sync_copy(key_hbm.at[pl.ds(off, n)],
                            key_v.at[pl.ds(0, n)])

            def unkey(i, _):
                fit_v[pl.ds(i * L, L)] = _from_key(key_v[pl.ds(i * L, L)])
                return 0
            lax.fori_loop(0, n // L, unkey, 0)
            pltpu.sync_copy(fit_v.at[pl.ds(0, n)],
                            fs_hbm.at[pl.ds(off, n)])

        @pl.when(c < NFULL)
        def _full():
            do(CHUNK)

        @pl.when(c == NFULL)
        def _tail():
            do(TAIL)


def kernel(x, fitness):
    bits = lax.bitcast_convert_type(fitness, jnp.int32)
    pad = lax.full((NP - N,), np.int32(0x7F800000), jnp.int32)
    k = jnp.concatenate([bits, pad])
    idx = None
    for p in range(3):
        h = _make_hist(SHIFTS[p], BITS[p], p == 0)(k)
        pref, tot = _make_scan(BITS[p])(h)
        if p == 0:
            k, idx = _make_permute(SHIFTS[p], BITS[p], True)(k, pref, tot)
        else:
            k, idx = _make_permute(SHIFTS[p], BITS[p], False)(k, idx, pref, tot)
    x_sorted, fitness_sorted = _gather_kernel(x, k, idx)
    return (x_sorted, fitness_sorted)
